# baseline (speedup 1.0000x reference)
import functools

import jax
import jax.numpy as jnp
from jax import lax
from jax.experimental import pallas as pl
from jax.experimental.pallas import tpu as pltpu
from jax.experimental.pallas import tpu_sc as plsc

DIM = 32
NEG = 5


def _make_sc_kernel(B):
    info = plsc.get_sparse_core_info()
    NC, NS, L = info.num_cores, info.num_subcores, info.num_lanes
    NW = NC * NS
    assert B % NW == 0 and DIM == 2 * L
    bpw = B // NW
    nbw = bpw * NEG

    mesh = plsc.VectorSubcoreMesh(core_axis_name="c", subcore_axis_name="s")

    @functools.partial(
        pl.kernel,
        mesh=mesh,
        out_type=jax.ShapeDtypeStruct((NW, DIM), jnp.float32),
        compiler_params=pltpu.CompilerParams(use_tc_tiling_on_sc=False),
        scratch_types=[
            pltpu.VMEM((bpw,), jnp.int32),
            pltpu.VMEM((bpw,), jnp.int32),
            pltpu.VMEM((bpw,), jnp.int32),
            pltpu.VMEM((bpw,), jnp.int32),
            pltpu.VMEM((nbw,), jnp.int32),
            pltpu.VMEM((nbw,), jnp.int32),
            pltpu.VMEM((bpw, DIM), jnp.float32),
            pltpu.VMEM((bpw, DIM), jnp.float32),
            pltpu.VMEM((bpw, DIM), jnp.float32),
            pltpu.VMEM((bpw, DIM), jnp.float32),
            pltpu.VMEM((nbw, DIM), jnp.float32),
            pltpu.VMEM((nbw, DIM), jnp.float32),
            pltpu.VMEM((DIM,), jnp.float32),
            pltpu.SemaphoreType.DMA,
        ],
    )
    def sc_kernel(rupos_h, aupos_h, rvpos_h, avpos_h, rnpos_h, anpos_h,
                  ruw_h, auw_h, rvw_h, avw_h, out_h,
                  iru, iau, irv, iav, irn, ian,
                  rru, rau, rrv, rav, rrn, ran, ostage, sem):
        wid = lax.axis_index("s") * NC + lax.axis_index("c")
        base = wid * bpw
        nbase = wid * nbw

        pltpu.sync_copy(rupos_h.at[pl.ds(base, bpw)], iru)
        pltpu.sync_copy(aupos_h.at[pl.ds(base, bpw)], iau)
        pltpu.sync_copy(rvpos_h.at[pl.ds(base, bpw)], irv)
        pltpu.sync_copy(avpos_h.at[pl.ds(base, bpw)], iav)
        pltpu.sync_copy(rnpos_h.at[pl.ds(nbase, nbw)], irn)
        pltpu.sync_copy(anpos_h.at[pl.ds(nbase, nbw)], ian)

        d0 = pltpu.async_copy(ruw_h.at[iru], rru, sem)
        d1 = pltpu.async_copy(auw_h.at[iau], rau, sem)
        d2 = pltpu.async_copy(rvw_h.at[irv], rrv, sem)
        d3 = pltpu.async_copy(avw_h.at[iav], rav, sem)
        d4 = pltpu.async_copy(rvw_h.at[irn], rrn, sem)
        d5 = pltpu.async_copy(avw_h.at[ian], ran, sem)
        d0.wait(); d1.wait(); d2.wait(); d3.wait(); d4.wait(); d5.wait()

        def body(i, carry):
            s0, s1, n0, n1 = carry
            u0 = rru[i, pl.ds(0, L)] + rau[i, pl.ds(0, L)]
            u1 = rru[i, pl.ds(L, L)] + rau[i, pl.ds(L, L)]
            v0 = rrv[i, pl.ds(0, L)] + rav[i, pl.ds(0, L)]
            v1 = rrv[i, pl.ds(L, L)] + rav[i, pl.ds(L, L)]
            s0 = s0 + u0 * v0
            s1 = s1 + u1 * v1
            j = i * NEG
            nv0 = rrn[j, pl.ds(0, L)] + ran[j, pl.ds(0, L)]
            nv1 = rrn[j, pl.ds(L, L)] + ran[j, pl.ds(L, L)]
            for n in range(1, NEG):
                nv0 = nv0 + rrn[j + n, pl.ds(0, L)] + ran[j + n, pl.ds(0, L)]
                nv1 = nv1 + rrn[j + n, pl.ds(L, L)] + ran[j + n, pl.ds(L, L)]
            n0 = n0 + u0 * nv0
            n1 = n1 + u1 * nv1
            return s0, s1, n0, n1

        z = jnp.zeros((L,), jnp.float32)
        s0, s1, n0, n1 = lax.fori_loop(0, bpw, body, (z, z, z, z))
        ostage[pl.ds(0, L)] = s0 + s1
        ostage[pl.ds(L, L)] = n0 + n1
        pltpu.sync_copy(ostage, out_h.at[wid])

    return sc_kernel, L


def kernel(upos, vpos, npos, ru_weight, rv_weight, au_weight, av_weight):
    B = upos.shape[1]
    sc_kernel, L = _make_sc_kernel(B)

    rupos, aupos = upos[0], upos[2]
    rvpos, avpos = vpos[0], vpos[2]
    rnpos = npos[0].reshape(-1)
    anpos = npos[2].reshape(-1)

    parts = sc_kernel(rupos, aupos, rvpos, avpos, rnpos, anpos,
                      ru_weight, au_weight, rv_weight, av_weight)
    score = jnp.sum(parts[:, :L])
    neg_score = jnp.sum(parts[:, L:])
    return jax.nn.log_sigmoid(score) + jax.nn.log_sigmoid(-neg_score)
